# baseline (speedup 1.0000x reference)
import functools

import jax
import jax.numpy as jnp
from jax import lax
from jax.experimental import pallas as pl
from jax.experimental.pallas import tpu as pltpu
from jax.experimental.pallas import tpu_sc as plsc

_V = 1000
_D = 1000
_B, _L = 1024, 50

_NC, _NS = 2, 16
_NW = _NC * _NS
_NB = 2
_BT = _B // _NW


def _make_gather():
    mesh = plsc.VectorSubcoreMesh(core_axis_name="c", subcore_axis_name="s")

    @functools.partial(
        pl.kernel,
        out_type=jax.ShapeDtypeStruct((_B, _L, _D), jnp.float32),
        mesh=mesh,
        scratch_types=[
            pltpu.VMEM((_BT, _L), jnp.int32),
            pltpu.VMEM((_NB, _L, _D), jnp.float32),
            [pltpu.SemaphoreType.DMA] * _NB,
            [pltpu.SemaphoreType.DMA] * _NB,
        ],
        compiler_params=pltpu.CompilerParams(use_tc_tiling_on_sc=False),
    )
    def gather_kernel(idx_hbm, table_hbm, out_hbm, idx_v, buf, gsems, ssems):
        wid = lax.axis_index("s") * _NC + lax.axis_index("c")
        base = wid * _BT
        pltpu.sync_copy(idx_hbm.at[wid], idx_v)

        def g_desc(c, b):
            return pltpu.make_async_copy(
                table_hbm.at[idx_v.at[c]], buf.at[b], gsems[b])

        def s_desc(c, b):
            return pltpu.make_async_copy(
                buf.at[b], out_hbm.at[base + c], ssems[b])

        g_desc(0, 0).start()

        def group(g, carry):
            c0 = g * _NB
            for b in range(_NB):
                c = c0 + b
                g_desc(c, b).wait()
                s_desc(c, b).start()
                f = c + 1
                bf = (b + 1) % _NB

                @pl.when(f < _BT)
                def _():
                    @pl.when(f >= _NB)
                    def _():
                        s_desc(f - _NB, bf).wait()
                    g_desc(f, bf).start()

            return carry

        lax.fori_loop(0, _BT // _NB, group, 0)
        s_desc(_BT - 2, (_BT - 2) % _NB).wait()
        s_desc(_BT - 1, (_BT - 1) % _NB).wait()

    return gather_kernel


_gather = _make_gather()


@jax.jit
def kernel(idx, logits_table):
    idx_w = idx.reshape(_NW, _BT, _L).astype(jnp.int32)
    return _gather(idx_w, logits_table)


# speedup vs baseline: 1.5110x; 1.5110x over previous
import functools

import jax
import jax.numpy as jnp
from jax import lax
from jax.experimental import pallas as pl
from jax.experimental.pallas import tpu as pltpu
from jax.experimental.pallas import tpu_sc as plsc

_V = 1000
_D = 1000
_DP = 1024
_B, _L = 1024, 50

_NC, _NS = 2, 16
_NW = _NC * _NS
_NB = 2
_BT = _B // _NW


def _make_gather():
    mesh = plsc.VectorSubcoreMesh(core_axis_name="c", subcore_axis_name="s")

    @functools.partial(
        pl.kernel,
        out_type=jax.ShapeDtypeStruct((_B, _L, _D), jnp.float32),
        mesh=mesh,
        scratch_types=[
            pltpu.VMEM((_BT, _L), jnp.int32),
            pltpu.VMEM((_NB, _L, 8, 128), jnp.float32),
            pltpu.VMEM((_NB, _L, 104), jnp.float32),
            [pltpu.SemaphoreType.DMA] * _NB,
            [pltpu.SemaphoreType.DMA] * _NB,
        ],
    )
    def gather_kernel(idx_hbm, table_hbm, out_hbm, idx_v, buf, tbuf, gsems,
                      ssems):
        wid = lax.axis_index("s") * _NC + lax.axis_index("c")
        base = wid * _BT
        pltpu.sync_copy(idx_hbm.at[wid], idx_v)

        def g_desc(c, b):
            return pltpu.make_async_copy(
                table_hbm.at[idx_v.at[c]], buf.at[b], gsems[b])

        def s_descs(c, b):
            ds = []
            for k in range(7):
                ds.append(pltpu.make_async_copy(
                    buf.at[b, :, k],
                    out_hbm.at[base + c, :, pl.ds(k * 128, 128)], ssems[b]))
            ds.append(pltpu.make_async_copy(
                tbuf.at[b], out_hbm.at[base + c, :, pl.ds(896, 104)],
                ssems[b]))
            return ds

        def fill_tail(b):
            def row(r, carry):
                for o in (0, 16, 32, 48, 64, 80, 88):
                    tbuf[b, r, pl.ds(o, 16)] = buf[b, r, 7, pl.ds(o, 16)]
                return carry

            lax.fori_loop(0, _L, row, 0)

        g_desc(0, 0).start()

        def group(g, carry):
            c0 = g * _NB
            for b in range(_NB):
                c = c0 + b
                g_desc(c, b).wait()
                fill_tail(b)
                for d in s_descs(c, b):
                    d.start()
                f = c + 1
                bf = (b + 1) % _NB

                @pl.when(f < _BT)
                def _():
                    @pl.when(f >= _NB)
                    def _():
                        for d in s_descs(f - _NB, bf):
                            d.wait()
                    g_desc(f, bf).start()

            return carry

        lax.fori_loop(0, _BT // _NB, group, 0)
        for c in (_BT - 2, _BT - 1):
            for d in s_descs(c, c % _NB):
                d.wait()

    return gather_kernel


_gather = _make_gather()


@jax.jit
def kernel(idx, logits_table):
    idx_w = idx.reshape(_NW, _BT, _L).astype(jnp.int32)
    table3 = jnp.pad(logits_table, ((0, 0), (0, _DP - _D))).reshape(_V, 8, 128)
    return _gather(idx_w, table3)
